# baseline (speedup 1.0000x reference)
import functools

import jax
import jax.numpy as jnp
from jax import lax
from jax.experimental import pallas as pl
from jax.experimental.pallas import tpu as pltpu
from jax.experimental.pallas import tpu_sc as plsc

N = 10000
E = 320000
F_IN = 128
H = 64
C3 = 3 * N
NC = 2
NS = 16
PAD3 = 30720
HALF = PAD3 // 2
PADE = 327680 - E
EP = E + PADE
CH = 128
NCHUNK = EP // CH
CPW2 = NCHUNK // NS
ZRH = HALF // NS
ZCH = 120
SENT = 2**31 - 1

_NEG_Q = 0.1
_POS_Q = 0.9


def _thresh_body(pred_ref, src_ref, dst_ref, ks0_ref, ks1_ref, kd0_ref,
                 kd1_ref):
    p = pred_ref[...]
    b = lax.bitcast_convert_type(p, jnp.uint32)
    top = jnp.uint32(0x80000000)
    ful = jnp.uint32(0xFFFFFFFF)
    key = jnp.where(b >= top, ful - b, b + top)

    sel_neg = p <= 0.0
    m_neg = jnp.sum(sel_neg.astype(jnp.int32))
    m_pos = jnp.int32(E) - m_neg

    def ranks(m_i, q):
        m_f = m_i.astype(jnp.float32)
        idx = q * (m_f - 1.0)
        lo_f = jnp.floor(idx)
        hw = idx - lo_f
        t_lo = jnp.clip(lo_f, 0.0, m_f - 1.0).astype(jnp.int32)
        t_hi = jnp.clip(jnp.ceil(idx), 0.0, m_f - 1.0).astype(jnp.int32)
        return t_lo, t_hi, hw

    tn_lo, tn_hi, hw_n = ranks(m_neg, _NEG_Q)
    tp_lo, tp_hi, hw_p = ranks(m_pos, _POS_Q)

    zero = jnp.uint32(0)
    targets = (tn_lo, tn_hi, tp_lo + m_neg, tp_hi + m_neg)

    def bs_body(_, st):
        los, his = st
        new_los, new_his = [], []
        for j in range(4):
            lo, hi = los[j], his[j]
            mid = lo + (hi - lo) // jnp.uint32(2)
            cnt = jnp.sum((key <= mid).astype(jnp.int32))
            good = cnt >= targets[j] + 1
            new_los.append(jnp.where(good, lo, mid + jnp.uint32(1)))
            new_his.append(jnp.where(good, mid, hi))
        return tuple(new_los), tuple(new_his)

    init = ((zero, zero, zero, zero), (ful, ful, ful, ful))
    (los, _) = lax.fori_loop(0, 32, bs_body, init)

    def unkey(k):
        bb = jnp.where(k >= top, k - top, ful - k)
        return lax.bitcast_convert_type(bb, jnp.float32)

    vn_lo, vn_hi, vp_lo, vp_hi = (unkey(k) for k in los)
    neg_thr = jnp.where(m_neg > 0, vn_lo * (1.0 - hw_n) + vn_hi * hw_n, 0.0)
    pos_thr = jnp.where(m_pos > 0, vp_lo * (1.0 - hw_p) + vp_hi * hw_p, 0.0)

    cls = jnp.where(p > pos_thr, 0, jnp.where(p < neg_thr, 2, 1)).astype(jnp.int32)
    ks = cls * N + src_ref[...]
    kd = cls * N + dst_ref[...]
    in0 = kd < HALF
    ks0_ref[...] = jnp.where(in0, ks, SENT)
    ks1_ref[...] = jnp.where(in0, SENT, ks)
    kd0_ref[...] = jnp.where(in0, kd, SENT)
    kd1_ref[...] = jnp.where(in0, SENT, kd - HALF)


def _thresh(pred2, src2, dst2):
    return pl.pallas_call(
        _thresh_body,
        out_shape=[jax.ShapeDtypeStruct((2500, 128), jnp.int32)] * 4,
    )(pred2, src2, dst2)


def _mlp_body(x_ref, w1_ref, b1_ref, w2_ref, b2_ref, h_ref):
    h1 = jnp.maximum(
        jnp.dot(x_ref[...], w1_ref[...], preferred_element_type=jnp.float32)
        + b1_ref[...], 0.0)
    h_ref[...] = jnp.maximum(
        jnp.dot(h1, w2_ref[...], preferred_element_type=jnp.float32)
        + b2_ref[...], 0.0)


def _mlp(feat, W1, b1r, W2, b2r):
    return pl.pallas_call(
        _mlp_body,
        grid=(10,),
        in_specs=[
            pl.BlockSpec((1000, F_IN), lambda i: (i, 0)),
            pl.BlockSpec((F_IN, H), lambda i: (0, 0)),
            pl.BlockSpec((1, H), lambda i: (0, 0)),
            pl.BlockSpec((H, H), lambda i: (0, 0)),
            pl.BlockSpec((1, H), lambda i: (0, 0)),
        ],
        out_specs=pl.BlockSpec((1000, H), lambda i: (i, 0)),
        out_shape=jax.ShapeDtypeStruct((N, H), jnp.float32),
    )(feat, W1, b1r, W2, b2r)


@functools.cache
def _sc_kernels():
    mesh = plsc.VectorSubcoreMesh(core_axis_name="c", subcore_axis_name="s")

    @functools.partial(
        pl.kernel,
        out_type=jax.ShapeDtypeStruct((PAD3,), jnp.float32),
        mesh=mesh,
        compiler_params=pltpu.CompilerParams(use_tc_tiling_on_sc=False),
        scratch_types=[
            pltpu.VMEM((CPW2, CH), jnp.int32),
            pltpu.VMEM((CH,), jnp.float32),
            pltpu.VMEM((ZRH,), jnp.float32),
            pltpu.VMEM_SHARED((HALF,), jnp.float32),
        ],
    )
    def _deg_sc(kd0_hbm, kd1_hbm, out_hbm, di2_v, ones_v, zer_v, acc_sh):
        c = lax.axis_index("c")
        s = lax.axis_index("s")

        def fill(ref, n, val):
            def fb(i, _):
                ref[pl.ds(i * 16, 16)] = jnp.full((16,), val, jnp.float32)
                return 0
            lax.fori_loop(0, n // 16, fb, 0)

        fill(ones_v, CH, 1.0)
        fill(zer_v, ZRH, 0.0)

        @pl.when(c == 0)
        def _():
            pltpu.sync_copy(kd0_hbm.at[pl.ds(s * CPW2, CPW2)], di2_v)

        @pl.when(c == 1)
        def _():
            pltpu.sync_copy(kd1_hbm.at[pl.ds(s * CPW2, CPW2)], di2_v)

        pltpu.sync_copy(zer_v, acc_sh.at[pl.ds(s * ZRH, ZRH)])
        plsc.subcore_barrier()

        def body(i, _):
            pltpu.sync_copy(
                ones_v,
                acc_sh.at[plsc.Indices(di2_v.at[i], ignored_value=SENT)],
                add=True)
            return 0

        lax.fori_loop(0, CPW2, body, 0)
        plsc.subcore_barrier()
        pltpu.sync_copy(acc_sh.at[pl.ds(s * ZRH, ZRH)],
                        out_hbm.at[pl.ds(c * HALF + s * ZRH, ZRH)])

    @functools.partial(
        pl.kernel,
        out_type=jax.ShapeDtypeStruct((PAD3, H), jnp.float32),
        mesh=mesh,
        compiler_params=pltpu.CompilerParams(use_tc_tiling_on_sc=False),
        scratch_types=[
            pltpu.VMEM((CPW2, CH), jnp.int32),
            pltpu.VMEM((CPW2, CH), jnp.int32),
            pltpu.VMEM((CH, H), jnp.float32),
            pltpu.VMEM((CH, H), jnp.float32),
            pltpu.VMEM((ZCH, H), jnp.float32),
            pltpu.VMEM_SHARED((HALF, H), jnp.float32),
            pltpu.SemaphoreType.DMA,
            pltpu.SemaphoreType.DMA,
        ],
    )
    def _hop_sc(gtab_hbm, ks0_hbm, ks1_hbm, kd0_hbm, kd1_hbm, out_hbm,
                si2_v, di2_v, rows0_v, rows1_v, zer_v,
                acc_sh, sem0, sem1):
        c = lax.axis_index("c")
        s = lax.axis_index("s")
        nz = H // 16
        rows = (rows0_v, rows1_v)
        sems = (sem0, sem1)

        def zb(i, _):
            zer_v[i // nz, pl.ds((i % nz) * 16, 16)] = jnp.zeros(
                (16,), jnp.float32)
            return 0

        lax.fori_loop(0, ZCH * nz, zb, 0)

        @pl.when(c == 0)
        def _():
            pltpu.sync_copy(ks0_hbm.at[pl.ds(s * CPW2, CPW2)], si2_v)
            pltpu.sync_copy(kd0_hbm.at[pl.ds(s * CPW2, CPW2)], di2_v)

        @pl.when(c == 1)
        def _():
            pltpu.sync_copy(ks1_hbm.at[pl.ds(s * CPW2, CPW2)], si2_v)
            pltpu.sync_copy(kd1_hbm.at[pl.ds(s * CPW2, CPW2)], di2_v)

        def zc(j, _):
            pltpu.sync_copy(zer_v, acc_sh.at[pl.ds(s * ZRH + j * ZCH, ZCH)])
            return 0

        lax.fori_loop(0, ZRH // ZCH, zc, 0)
        plsc.subcore_barrier()

        def body(j, _):
            cps = []
            for k in range(2):
                cps.append(pltpu.async_copy(
                    gtab_hbm.at[plsc.Indices(si2_v.at[2 * j + k],
                                             ignored_value=SENT)],
                    rows[k], sems[k]))
            for k in range(2):
                cps[k].wait()
                pltpu.sync_copy(
                    rows[k],
                    acc_sh.at[plsc.Indices(di2_v.at[2 * j + k],
                                           ignored_value=SENT)],
                    add=True)
            return 0

        lax.fori_loop(0, CPW2 // 2, body, 0)
        plsc.subcore_barrier()
        pltpu.sync_copy(acc_sh.at[pl.ds(s * ZRH, ZRH)],
                        out_hbm.at[pl.ds(c * HALF + s * ZRH, ZRH)])

    return _deg_sc, _hop_sc


def _build_body(degp_ref, h_ref, dinv_ref, g1_ref):
    dinv = lax.rsqrt(jnp.maximum(degp_ref[...], 1.0))
    dinv_ref[...] = dinv
    g1_ref[...] = h_ref[...] * dinv


def _build(degp, h):
    return pl.pallas_call(
        _build_body,
        grid=(3, 10),
        in_specs=[
            pl.BlockSpec((1000, 1), lambda c, i: (c * 10 + i, 0)),
            pl.BlockSpec((1000, H), lambda c, i: (i, 0)),
        ],
        out_specs=[
            pl.BlockSpec((1000, 1), lambda c, i: (c * 10 + i, 0)),
            pl.BlockSpec((1000, H), lambda c, i: (c * 10 + i, 0)),
        ],
        out_shape=[
            jax.ShapeDtypeStruct((C3, 1), jnp.float32),
            jax.ShapeDtypeStruct((PAD3, H), jnp.float32),
        ],
    )(degp, h)


def _comb1_body(a1_ref, h_ref, dinv_ref, f1_ref, g2_ref):
    dinv = dinv_ref[...]
    f1 = h_ref[...] - a1_ref[...] * dinv
    f1_ref[...] = f1
    g2_ref[...] = f1 * dinv


def _comb1(a1, h, dinv):
    return pl.pallas_call(
        _comb1_body,
        grid=(3, 10),
        in_specs=[
            pl.BlockSpec((1000, H), lambda c, i: (c * 10 + i, 0)),
            pl.BlockSpec((1000, H), lambda c, i: (i, 0)),
            pl.BlockSpec((1000, 1), lambda c, i: (c * 10 + i, 0)),
        ],
        out_specs=[
            pl.BlockSpec((1000, H), lambda c, i: (c * 10 + i, 0)),
            pl.BlockSpec((1000, H), lambda c, i: (c * 10 + i, 0)),
        ],
        out_shape=[
            jax.ShapeDtypeStruct((C3, H), jnp.float32),
            jax.ShapeDtypeStruct((PAD3, H), jnp.float32),
        ],
    )(a1, h, dinv)


_TH0 = (0.0, 0.0, 3.0)
_TH1 = (0.0, 3.0, -3.0)
_TH2 = (0.75, -1.5, 0.75)


def _final_body(h_ref, f1a_ref, f1b_ref, f1c_ref, a2a_ref, a2b_ref, a2c_ref,
                dva_ref, dvb_ref, dvc_ref, w3_ref, b3_ref, out_ref):
    h = h_ref[...]
    acc = jnp.broadcast_to(b3_ref[...], (1000, H))
    for cc, (f1_ref, a2_ref, dv_ref) in enumerate(
        ((f1a_ref, a2a_ref, dva_ref),
         (f1b_ref, a2b_ref, dvb_ref),
         (f1c_ref, a2c_ref, dvc_ref))
    ):
        f1 = f1_ref[...]
        f2 = f1 - a2_ref[...] * dv_ref[...]
        part = _TH0[cc] * h + _TH1[cc] * f1 + _TH2[cc] * f2
        acc = acc + jnp.dot(part, w3_ref[cc],
                            preferred_element_type=jnp.float32)
    out_ref[...] = jnp.maximum(acc, 0.0)


def _final(h, f1, a2, dinv, W3r, b3r):
    f1_spec = lambda c: pl.BlockSpec((1000, H), lambda i, c=c: (c * 10 + i, 0))
    dv_spec = lambda c: pl.BlockSpec((1000, 1), lambda i, c=c: (c * 10 + i, 0))
    return pl.pallas_call(
        _final_body,
        grid=(10,),
        in_specs=[
            pl.BlockSpec((1000, H), lambda i: (i, 0)),
            f1_spec(0), f1_spec(1), f1_spec(2),
            f1_spec(0), f1_spec(1), f1_spec(2),
            dv_spec(0), dv_spec(1), dv_spec(2),
            pl.BlockSpec((3, H, H), lambda i: (0, 0, 0)),
            pl.BlockSpec((1, H), lambda i: (0, 0)),
        ],
        out_specs=pl.BlockSpec((1000, H), lambda i: (i, 0)),
        out_shape=jax.ShapeDtypeStruct((N, H), jnp.float32),
    )(h, f1, f1, f1, a2, a2, a2, dinv, dinv, dinv, W3r, b3r)


def kernel(feat, edge_index, edge_pred, W1, b1, W2, b2, W3, b3):
    pred2 = edge_pred.reshape(2500, 128)
    src2 = edge_index[0].reshape(2500, 128)
    dst2 = edge_index[1].reshape(2500, 128)
    ks02, ks12, kd02, kd12 = _thresh(pred2, src2, dst2)
    pads = jnp.full((PADE,), SENT, jnp.int32)
    ks0 = jnp.concatenate([ks02.reshape(E), pads]).reshape(NCHUNK, CH)
    ks1 = jnp.concatenate([ks12.reshape(E), pads]).reshape(NCHUNK, CH)
    kd0 = jnp.concatenate([kd02.reshape(E), pads]).reshape(NCHUNK, CH)
    kd1 = jnp.concatenate([kd12.reshape(E), pads]).reshape(NCHUNK, CH)

    h = _mlp(feat, W1, b1.reshape(1, H), W2, b2.reshape(1, H))

    _deg_sc, _hop_sc = _sc_kernels()
    degp = _deg_sc(kd0, kd1)[:, None]
    dinv, G1 = _build(degp, h)
    A1 = _hop_sc(G1, ks0, ks1, kd0, kd1)
    f1, G2 = _comb1(A1, h, dinv)
    A2 = _hop_sc(G2, ks0, ks1, kd0, kd1)
    W3r = W3.reshape(3, H, H)
    return _final(h, f1, A2, dinv, W3r, b3.reshape(1, H))


# speedup vs baseline: 1.0852x; 1.0852x over previous
import functools

import jax
import jax.numpy as jnp
from jax import lax
from jax.experimental import pallas as pl
from jax.experimental.pallas import tpu as pltpu
from jax.experimental.pallas import tpu_sc as plsc

N = 10000
E = 320000
F_IN = 128
H = 64
C3 = 3 * N
NC = 2
NS = 16
PAD3 = 30720
HALF = PAD3 // 2
PADE = 327680 - E
EP = E + PADE
CH = 128
NCHUNK = EP // CH
CPW2 = NCHUNK // NS
ZRH = HALF // NS
ZCH = 120
SENT = 2**31 - 1

_NEG_Q = 0.1
_POS_Q = 0.9


def _thresh_body(pred_ref, src_ref, dst_ref, ksrc_ref, kd0_ref, kd1_ref):
    p = pred_ref[...]
    b = lax.bitcast_convert_type(p, jnp.uint32)
    top = jnp.uint32(0x80000000)
    ful = jnp.uint32(0xFFFFFFFF)
    key = jnp.where(b >= top, ful - b, b + top)

    sel_neg = p <= 0.0
    m_neg = jnp.sum(sel_neg.astype(jnp.int32))
    m_pos = jnp.int32(E) - m_neg

    def ranks(m_i, q):
        m_f = m_i.astype(jnp.float32)
        idx = q * (m_f - 1.0)
        lo_f = jnp.floor(idx)
        hw = idx - lo_f
        t_lo = jnp.clip(lo_f, 0.0, m_f - 1.0).astype(jnp.int32)
        t_hi = jnp.clip(jnp.ceil(idx), 0.0, m_f - 1.0).astype(jnp.int32)
        return t_lo, t_hi, hw

    tn_lo, tn_hi, hw_n = ranks(m_neg, _NEG_Q)
    tp_lo, tp_hi, hw_p = ranks(m_pos, _POS_Q)

    zero = jnp.uint32(0)
    targets = (tn_lo, tn_hi, tp_lo + m_neg, tp_hi + m_neg)

    def bs_body(_, st):
        los, his = st
        new_los, new_his = [], []
        for j in range(4):
            lo, hi = los[j], his[j]
            mid = lo + (hi - lo) // jnp.uint32(2)
            cnt = jnp.sum((key <= mid).astype(jnp.int32))
            good = cnt >= targets[j] + 1
            new_los.append(jnp.where(good, lo, mid + jnp.uint32(1)))
            new_his.append(jnp.where(good, mid, hi))
        return tuple(new_los), tuple(new_his)

    init = ((zero, zero, zero, zero), (ful, ful, ful, ful))
    (los, _) = lax.fori_loop(0, 32, bs_body, init)

    def unkey(k):
        bb = jnp.where(k >= top, k - top, ful - k)
        return lax.bitcast_convert_type(bb, jnp.float32)

    vn_lo, vn_hi, vp_lo, vp_hi = (unkey(k) for k in los)
    neg_thr = jnp.where(m_neg > 0, vn_lo * (1.0 - hw_n) + vn_hi * hw_n, 0.0)
    pos_thr = jnp.where(m_pos > 0, vp_lo * (1.0 - hw_p) + vp_hi * hw_p, 0.0)

    cls = jnp.where(p > pos_thr, 0, jnp.where(p < neg_thr, 2, 1)).astype(jnp.int32)
    ksrc_ref[...] = cls * N + src_ref[...]
    kd = cls * N + dst_ref[...]
    in0 = kd < HALF
    kd0_ref[...] = jnp.where(in0, kd, SENT)
    kd1_ref[...] = jnp.where(in0, SENT, kd - HALF)


def _thresh(pred2, src2, dst2):
    return pl.pallas_call(
        _thresh_body,
        out_shape=[jax.ShapeDtypeStruct((2500, 128), jnp.int32)] * 3,
    )(pred2, src2, dst2)


def _mlp_body(x_ref, w1_ref, b1_ref, w2_ref, b2_ref, h_ref):
    h1 = jnp.maximum(
        jnp.dot(x_ref[...], w1_ref[...], preferred_element_type=jnp.float32)
        + b1_ref[...], 0.0)
    h_ref[...] = jnp.maximum(
        jnp.dot(h1, w2_ref[...], preferred_element_type=jnp.float32)
        + b2_ref[...], 0.0)


def _mlp(feat, W1, b1r, W2, b2r):
    return pl.pallas_call(
        _mlp_body,
        grid=(10,),
        in_specs=[
            pl.BlockSpec((1000, F_IN), lambda i: (i, 0)),
            pl.BlockSpec((F_IN, H), lambda i: (0, 0)),
            pl.BlockSpec((1, H), lambda i: (0, 0)),
            pl.BlockSpec((H, H), lambda i: (0, 0)),
            pl.BlockSpec((1, H), lambda i: (0, 0)),
        ],
        out_specs=pl.BlockSpec((1000, H), lambda i: (i, 0)),
        out_shape=jax.ShapeDtypeStruct((N, H), jnp.float32),
    )(feat, W1, b1r, W2, b2r)


@functools.cache
def _sc_kernels():
    mesh = plsc.VectorSubcoreMesh(core_axis_name="c", subcore_axis_name="s")

    @functools.partial(
        pl.kernel,
        out_type=jax.ShapeDtypeStruct((PAD3,), jnp.float32),
        mesh=mesh,
        compiler_params=pltpu.CompilerParams(use_tc_tiling_on_sc=False),
        scratch_types=[
            pltpu.VMEM((CPW2, CH), jnp.int32),
            pltpu.VMEM((CH,), jnp.float32),
            pltpu.VMEM((ZRH,), jnp.float32),
            pltpu.VMEM_SHARED((HALF,), jnp.float32),
        ],
    )
    def _deg_sc(kd0_hbm, kd1_hbm, out_hbm, di2_v, ones_v, zer_v, acc_sh):
        c = lax.axis_index("c")
        s = lax.axis_index("s")

        def fill(ref, n, val):
            def fb(i, _):
                ref[pl.ds(i * 16, 16)] = jnp.full((16,), val, jnp.float32)
                return 0
            lax.fori_loop(0, n // 16, fb, 0)

        fill(ones_v, CH, 1.0)
        fill(zer_v, ZRH, 0.0)

        @pl.when(c == 0)
        def _():
            pltpu.sync_copy(kd0_hbm.at[pl.ds(s * CPW2, CPW2)], di2_v)

        @pl.when(c == 1)
        def _():
            pltpu.sync_copy(kd1_hbm.at[pl.ds(s * CPW2, CPW2)], di2_v)

        pltpu.sync_copy(zer_v, acc_sh.at[pl.ds(s * ZRH, ZRH)])
        plsc.subcore_barrier()

        def body(i, _):
            pltpu.sync_copy(
                ones_v,
                acc_sh.at[plsc.Indices(di2_v.at[i], ignored_value=SENT)],
                add=True)
            return 0

        lax.fori_loop(0, CPW2, body, 0)
        plsc.subcore_barrier()
        pltpu.sync_copy(acc_sh.at[pl.ds(s * ZRH, ZRH)],
                        out_hbm.at[pl.ds(c * HALF + s * ZRH, ZRH)])

    @functools.partial(
        pl.kernel,
        out_type=jax.ShapeDtypeStruct((PAD3, H), jnp.float32),
        mesh=mesh,
        compiler_params=pltpu.CompilerParams(use_tc_tiling_on_sc=False),
        scratch_types=[
            pltpu.VMEM((CPW2, CH), jnp.int32),
            pltpu.VMEM((CPW2, CH), jnp.int32),
            pltpu.VMEM((CH, H), jnp.float32),
            pltpu.VMEM((CH, H), jnp.float32),
            pltpu.VMEM((ZCH, H), jnp.float32),
            pltpu.VMEM_SHARED((HALF, H), jnp.float32),
            pltpu.SemaphoreType.DMA,
            pltpu.SemaphoreType.DMA,
        ],
    )
    def _hop_sc(gtab_hbm, ksrc_hbm, kd0_hbm, kd1_hbm, out_hbm,
                si2_v, di2_v, rows0_v, rows1_v, zer_v, acc_sh, sem0, sem1):
        c = lax.axis_index("c")
        s = lax.axis_index("s")
        nz = H // 16

        def zb(i, _):
            zer_v[i // nz, pl.ds((i % nz) * 16, 16)] = jnp.zeros(
                (16,), jnp.float32)
            return 0

        lax.fori_loop(0, ZCH * nz, zb, 0)
        pltpu.sync_copy(ksrc_hbm.at[pl.ds(s * CPW2, CPW2)], si2_v)

        @pl.when(c == 0)
        def _():
            pltpu.sync_copy(kd0_hbm.at[pl.ds(s * CPW2, CPW2)], di2_v)

        @pl.when(c == 1)
        def _():
            pltpu.sync_copy(kd1_hbm.at[pl.ds(s * CPW2, CPW2)], di2_v)

        def zc(j, _):
            pltpu.sync_copy(zer_v, acc_sh.at[pl.ds(s * ZRH + j * ZCH, ZCH)])
            return 0

        lax.fori_loop(0, ZRH // ZCH, zc, 0)
        plsc.subcore_barrier()

        def wait_rows(rows_v, sem):
            pltpu.make_async_copy(
                gtab_hbm.at[pl.ds(0, CH)], rows_v, sem).wait()

        def scat(rows_v, i):
            pltpu.sync_copy(
                rows_v,
                acc_sh.at[plsc.Indices(di2_v.at[i], ignored_value=SENT)],
                add=True)

        pltpu.async_copy(gtab_hbm.at[si2_v.at[0]], rows0_v, sem0)

        def body(j, _):
            i0 = 2 * j
            i1 = i0 + 1
            pltpu.async_copy(gtab_hbm.at[si2_v.at[i1]], rows1_v, sem1)
            wait_rows(rows0_v, sem0)
            scat(rows0_v, i0)
            inext = jnp.minimum(i1 + 1, CPW2 - 1)
            pltpu.async_copy(gtab_hbm.at[si2_v.at[inext]], rows0_v, sem0)
            wait_rows(rows1_v, sem1)
            scat(rows1_v, i1)
            return 0

        lax.fori_loop(0, CPW2 // 2, body, 0)
        wait_rows(rows0_v, sem0)
        plsc.subcore_barrier()
        pltpu.sync_copy(acc_sh.at[pl.ds(s * ZRH, ZRH)],
                        out_hbm.at[pl.ds(c * HALF + s * ZRH, ZRH)])

    return _deg_sc, _hop_sc


def _build_body(degp_ref, h_ref, dinv_ref, g1_ref):
    dinv = lax.rsqrt(jnp.maximum(degp_ref[...], 1.0))
    dinv_ref[...] = dinv
    g1_ref[...] = h_ref[...] * dinv


def _build(degp, h):
    return pl.pallas_call(
        _build_body,
        grid=(3, 10),
        in_specs=[
            pl.BlockSpec((1000, 1), lambda c, i: (c * 10 + i, 0)),
            pl.BlockSpec((1000, H), lambda c, i: (i, 0)),
        ],
        out_specs=[
            pl.BlockSpec((1000, 1), lambda c, i: (c * 10 + i, 0)),
            pl.BlockSpec((1000, H), lambda c, i: (c * 10 + i, 0)),
        ],
        out_shape=[
            jax.ShapeDtypeStruct((C3, 1), jnp.float32),
            jax.ShapeDtypeStruct((PAD3, H), jnp.float32),
        ],
    )(degp, h)


def _comb1_body(a1_ref, h_ref, dinv_ref, f1_ref, g2_ref):
    dinv = dinv_ref[...]
    f1 = h_ref[...] - a1_ref[...] * dinv
    f1_ref[...] = f1
    g2_ref[...] = f1 * dinv


def _comb1(a1, h, dinv):
    return pl.pallas_call(
        _comb1_body,
        grid=(3, 10),
        in_specs=[
            pl.BlockSpec((1000, H), lambda c, i: (c * 10 + i, 0)),
            pl.BlockSpec((1000, H), lambda c, i: (i, 0)),
            pl.BlockSpec((1000, 1), lambda c, i: (c * 10 + i, 0)),
        ],
        out_specs=[
            pl.BlockSpec((1000, H), lambda c, i: (c * 10 + i, 0)),
            pl.BlockSpec((1000, H), lambda c, i: (c * 10 + i, 0)),
        ],
        out_shape=[
            jax.ShapeDtypeStruct((C3, H), jnp.float32),
            jax.ShapeDtypeStruct((PAD3, H), jnp.float32),
        ],
    )(a1, h, dinv)


_TH0 = (0.0, 0.0, 3.0)
_TH1 = (0.0, 3.0, -3.0)
_TH2 = (0.75, -1.5, 0.75)


def _final_body(h_ref, f1a_ref, f1b_ref, f1c_ref, a2a_ref, a2b_ref, a2c_ref,
                dva_ref, dvb_ref, dvc_ref, w3_ref, b3_ref, out_ref):
    h = h_ref[...]
    acc = jnp.broadcast_to(b3_ref[...], (1000, H))
    for cc, (f1_ref, a2_ref, dv_ref) in enumerate(
        ((f1a_ref, a2a_ref, dva_ref),
         (f1b_ref, a2b_ref, dvb_ref),
         (f1c_ref, a2c_ref, dvc_ref))
    ):
        f1 = f1_ref[...]
        f2 = f1 - a2_ref[...] * dv_ref[...]
        part = _TH0[cc] * h + _TH1[cc] * f1 + _TH2[cc] * f2
        acc = acc + jnp.dot(part, w3_ref[cc],
                            preferred_element_type=jnp.float32)
    out_ref[...] = jnp.maximum(acc, 0.0)


def _final(h, f1, a2, dinv, W3r, b3r):
    f1_spec = lambda c: pl.BlockSpec((1000, H), lambda i, c=c: (c * 10 + i, 0))
    dv_spec = lambda c: pl.BlockSpec((1000, 1), lambda i, c=c: (c * 10 + i, 0))
    return pl.pallas_call(
        _final_body,
        grid=(10,),
        in_specs=[
            pl.BlockSpec((1000, H), lambda i: (i, 0)),
            f1_spec(0), f1_spec(1), f1_spec(2),
            f1_spec(0), f1_spec(1), f1_spec(2),
            dv_spec(0), dv_spec(1), dv_spec(2),
            pl.BlockSpec((3, H, H), lambda i: (0, 0, 0)),
            pl.BlockSpec((1, H), lambda i: (0, 0)),
        ],
        out_specs=pl.BlockSpec((1000, H), lambda i: (i, 0)),
        out_shape=jax.ShapeDtypeStruct((N, H), jnp.float32),
    )(h, f1, f1, f1, a2, a2, a2, dinv, dinv, dinv, W3r, b3r)


def kernel(feat, edge_index, edge_pred, W1, b1, W2, b2, W3, b3):
    pred2 = edge_pred.reshape(2500, 128)
    src2 = edge_index[0].reshape(2500, 128)
    dst2 = edge_index[1].reshape(2500, 128)
    ksrc2, kd02, kd12 = _thresh(pred2, src2, dst2)
    padg = C3 + (jnp.arange(PADE, dtype=jnp.int32) % (PAD3 - C3))
    pads = jnp.full((PADE,), SENT, jnp.int32)
    ksrc = jnp.concatenate([ksrc2.reshape(E), padg]).reshape(NCHUNK, CH)
    kd0 = jnp.concatenate([kd02.reshape(E), pads]).reshape(NCHUNK, CH)
    kd1 = jnp.concatenate([kd12.reshape(E), pads]).reshape(NCHUNK, CH)

    h = _mlp(feat, W1, b1.reshape(1, H), W2, b2.reshape(1, H))

    _deg_sc, _hop_sc = _sc_kernels()
    degp = _deg_sc(kd0, kd1)[:, None]
    dinv, G1 = _build(degp, h)
    A1 = _hop_sc(G1, ksrc, kd0, kd1)
    f1, G2 = _comb1(A1, h, dinv)
    A2 = _hop_sc(G2, ksrc, kd0, kd1)
    W3r = W3.reshape(3, H, H)
    return _final(h, f1, A2, dinv, W3r, b3.reshape(1, H))


# speedup vs baseline: 1.1239x; 1.0357x over previous
import functools

import jax
import jax.numpy as jnp
from jax import lax
from jax.experimental import pallas as pl
from jax.experimental.pallas import tpu as pltpu
from jax.experimental.pallas import tpu_sc as plsc

N = 10000
E = 320000
F_IN = 128
H = 64
C3 = 3 * N
NC = 2
NS = 16
PAD3 = 30720
HALF = PAD3 // 2
PADE = 327680 - E
EP = E + PADE
CH = 128
NCHUNK = EP // CH
CPW2 = NCHUNK // NS
ZRH = HALF // NS
ZCH = 120
SENT = 2**31 - 1

_NEG_Q = 0.1
_POS_Q = 0.9


def _thresh_body(pred_ref, src_ref, dst_ref, ks0_ref, ks1_ref, kd0_ref,
                 kd1_ref):
    p = pred_ref[...]
    b = lax.bitcast_convert_type(p, jnp.uint32)
    top = jnp.uint32(0x80000000)
    ful = jnp.uint32(0xFFFFFFFF)
    key = jnp.where(b >= top, ful - b, b + top)

    sel_neg = p <= 0.0
    m_neg = jnp.sum(sel_neg.astype(jnp.int32))
    m_pos = jnp.int32(E) - m_neg

    def ranks(m_i, q):
        m_f = m_i.astype(jnp.float32)
        idx = q * (m_f - 1.0)
        lo_f = jnp.floor(idx)
        hw = idx - lo_f
        t_lo = jnp.clip(lo_f, 0.0, m_f - 1.0).astype(jnp.int32)
        t_hi = jnp.clip(jnp.ceil(idx), 0.0, m_f - 1.0).astype(jnp.int32)
        return t_lo, t_hi, hw

    tn_lo, tn_hi, hw_n = ranks(m_neg, _NEG_Q)
    tp_lo, tp_hi, hw_p = ranks(m_pos, _POS_Q)

    zero = jnp.uint32(0)
    targets = (tn_lo, tn_hi, tp_lo + m_neg, tp_hi + m_neg)

    def bs_body(_, st):
        los, his = st
        new_los, new_his = [], []
        for j in range(4):
            lo, hi = los[j], his[j]
            mid = lo + (hi - lo) // jnp.uint32(2)
            cnt = jnp.sum((key <= mid).astype(jnp.int32))
            good = cnt >= targets[j] + 1
            new_los.append(jnp.where(good, lo, mid + jnp.uint32(1)))
            new_his.append(jnp.where(good, mid, hi))
        return tuple(new_los), tuple(new_his)

    init = ((zero, zero, zero, zero), (ful, ful, ful, ful))
    (los, _) = lax.fori_loop(0, 32, bs_body, init)

    def unkey(k):
        bb = jnp.where(k >= top, k - top, ful - k)
        return lax.bitcast_convert_type(bb, jnp.float32)

    vn_lo, vn_hi, vp_lo, vp_hi = (unkey(k) for k in los)
    neg_thr = jnp.where(m_neg > 0, vn_lo * (1.0 - hw_n) + vn_hi * hw_n, 0.0)
    pos_thr = jnp.where(m_pos > 0, vp_lo * (1.0 - hw_p) + vp_hi * hw_p, 0.0)

    cls = jnp.where(p > pos_thr, 0, jnp.where(p < neg_thr, 2, 1)).astype(jnp.int32)
    ks = cls * N + src_ref[...]
    kd = cls * N + dst_ref[...]
    in0 = kd < HALF
    ks0_ref[...] = jnp.where(in0, ks, SENT)
    ks1_ref[...] = jnp.where(in0, SENT, ks)
    kd0_ref[...] = jnp.where(in0, kd, SENT)
    kd1_ref[...] = jnp.where(in0, SENT, kd - HALF)


def _thresh(pred2, src2, dst2):
    return pl.pallas_call(
        _thresh_body,
        out_shape=[jax.ShapeDtypeStruct((2500, 128), jnp.int32)] * 4,
    )(pred2, src2, dst2)


def _mlp_body(x_ref, w1_ref, b1_ref, w2_ref, b2_ref, h_ref):
    h1 = jnp.maximum(
        jnp.dot(x_ref[...], w1_ref[...], preferred_element_type=jnp.float32)
        + b1_ref[...], 0.0)
    h_ref[...] = jnp.maximum(
        jnp.dot(h1, w2_ref[...], preferred_element_type=jnp.float32)
        + b2_ref[...], 0.0)


def _mlp(feat, W1, b1r, W2, b2r):
    return pl.pallas_call(
        _mlp_body,
        grid=(10,),
        in_specs=[
            pl.BlockSpec((1000, F_IN), lambda i: (i, 0)),
            pl.BlockSpec((F_IN, H), lambda i: (0, 0)),
            pl.BlockSpec((1, H), lambda i: (0, 0)),
            pl.BlockSpec((H, H), lambda i: (0, 0)),
            pl.BlockSpec((1, H), lambda i: (0, 0)),
        ],
        out_specs=pl.BlockSpec((1000, H), lambda i: (i, 0)),
        out_shape=jax.ShapeDtypeStruct((N, H), jnp.float32),
    )(feat, W1, b1r, W2, b2r)


@functools.cache
def _sc_kernels():
    mesh = plsc.VectorSubcoreMesh(core_axis_name="c", subcore_axis_name="s")

    @functools.partial(
        pl.kernel,
        out_type=jax.ShapeDtypeStruct((PAD3,), jnp.float32),
        mesh=mesh,
        compiler_params=pltpu.CompilerParams(use_tc_tiling_on_sc=False),
        scratch_types=[
            pltpu.VMEM((CPW2, CH), jnp.int32),
            pltpu.VMEM((CH,), jnp.float32),
            pltpu.VMEM((ZRH,), jnp.float32),
            pltpu.VMEM_SHARED((HALF,), jnp.float32),
        ],
    )
    def _deg_sc(kd0_hbm, kd1_hbm, out_hbm, di2_v, ones_v, zer_v, acc_sh):
        c = lax.axis_index("c")
        s = lax.axis_index("s")

        def fill(ref, n, val):
            def fb(i, _):
                ref[pl.ds(i * 16, 16)] = jnp.full((16,), val, jnp.float32)
                return 0
            lax.fori_loop(0, n // 16, fb, 0)

        fill(ones_v, CH, 1.0)
        fill(zer_v, ZRH, 0.0)

        @pl.when(c == 0)
        def _():
            pltpu.sync_copy(kd0_hbm.at[pl.ds(s * CPW2, CPW2)], di2_v)

        @pl.when(c == 1)
        def _():
            pltpu.sync_copy(kd1_hbm.at[pl.ds(s * CPW2, CPW2)], di2_v)

        pltpu.sync_copy(zer_v, acc_sh.at[pl.ds(s * ZRH, ZRH)])
        plsc.subcore_barrier()

        def body(i, _):
            pltpu.sync_copy(
                ones_v,
                acc_sh.at[plsc.Indices(di2_v.at[i], ignored_value=SENT)],
                add=True)
            return 0

        lax.fori_loop(0, CPW2, body, 0)
        plsc.subcore_barrier()
        pltpu.sync_copy(acc_sh.at[pl.ds(s * ZRH, ZRH)],
                        out_hbm.at[pl.ds(c * HALF + s * ZRH, ZRH)])

    @functools.partial(
        pl.kernel,
        out_type=jax.ShapeDtypeStruct((PAD3, H), jnp.float32),
        mesh=mesh,
        compiler_params=pltpu.CompilerParams(use_tc_tiling_on_sc=False),
        scratch_types=[
            pltpu.VMEM((CPW2, CH), jnp.int32),
            pltpu.VMEM((CPW2, CH), jnp.int32),
            pltpu.VMEM((CH, H), jnp.float32),
            pltpu.VMEM((CH, H), jnp.float32),
            pltpu.VMEM((ZCH, H), jnp.float32),
            pltpu.VMEM_SHARED((HALF, H), jnp.float32),
            pltpu.SemaphoreType.DMA,
            pltpu.SemaphoreType.DMA,
        ],
    )
    def _hop_sc(gtab_hbm, ks0_hbm, ks1_hbm, kd0_hbm, kd1_hbm, out_hbm,
                si2_v, di2_v, rows0_v, rows1_v, zer_v, acc_sh, sem0, sem1):
        c = lax.axis_index("c")
        s = lax.axis_index("s")
        nz = H // 16

        def zb(i, _):
            zer_v[i // nz, pl.ds((i % nz) * 16, 16)] = jnp.zeros(
                (16,), jnp.float32)
            return 0

        lax.fori_loop(0, ZCH * nz, zb, 0)

        @pl.when(c == 0)
        def _():
            pltpu.sync_copy(ks0_hbm.at[pl.ds(s * CPW2, CPW2)], si2_v)
            pltpu.sync_copy(kd0_hbm.at[pl.ds(s * CPW2, CPW2)], di2_v)

        @pl.when(c == 1)
        def _():
            pltpu.sync_copy(ks1_hbm.at[pl.ds(s * CPW2, CPW2)], si2_v)
            pltpu.sync_copy(kd1_hbm.at[pl.ds(s * CPW2, CPW2)], di2_v)

        def zc(j, _):
            pltpu.sync_copy(zer_v, acc_sh.at[pl.ds(s * ZRH + j * ZCH, ZCH)])
            return 0

        lax.fori_loop(0, ZRH // ZCH, zc, 0)
        plsc.subcore_barrier()

        def gath(i, rows_v, sem):
            pltpu.async_copy(
                gtab_hbm.at[plsc.Indices(si2_v.at[i], ignored_value=SENT)],
                rows_v, sem)

        def wait_rows(rows_v, sem):
            pltpu.make_async_copy(
                gtab_hbm.at[plsc.Indices(si2_v.at[0], ignored_value=SENT)],
                rows_v, sem).wait()

        def scat(rows_v, i):
            pltpu.sync_copy(
                rows_v,
                acc_sh.at[plsc.Indices(di2_v.at[i], ignored_value=SENT)],
                add=True)

        gath(0, rows0_v, sem0)

        def body(j, _):
            i0 = 2 * j
            i1 = i0 + 1
            gath(i1, rows1_v, sem1)
            wait_rows(rows0_v, sem0)
            scat(rows0_v, i0)
            inext = jnp.minimum(i1 + 1, CPW2 - 1)
            gath(inext, rows0_v, sem0)
            wait_rows(rows1_v, sem1)
            scat(rows1_v, i1)
            return 0

        lax.fori_loop(0, CPW2 // 2, body, 0)
        wait_rows(rows0_v, sem0)
        plsc.subcore_barrier()
        pltpu.sync_copy(acc_sh.at[pl.ds(s * ZRH, ZRH)],
                        out_hbm.at[pl.ds(c * HALF + s * ZRH, ZRH)])

    return _deg_sc, _hop_sc


def _build_body(degp_ref, h_ref, dinv_ref, g1_ref):
    dinv = lax.rsqrt(jnp.maximum(degp_ref[...], 1.0))
    dinv_ref[...] = dinv
    g1_ref[...] = h_ref[...] * dinv


def _build(degp, h):
    return pl.pallas_call(
        _build_body,
        grid=(3, 10),
        in_specs=[
            pl.BlockSpec((1000, 1), lambda c, i: (c * 10 + i, 0)),
            pl.BlockSpec((1000, H), lambda c, i: (i, 0)),
        ],
        out_specs=[
            pl.BlockSpec((1000, 1), lambda c, i: (c * 10 + i, 0)),
            pl.BlockSpec((1000, H), lambda c, i: (c * 10 + i, 0)),
        ],
        out_shape=[
            jax.ShapeDtypeStruct((C3, 1), jnp.float32),
            jax.ShapeDtypeStruct((PAD3, H), jnp.float32),
        ],
    )(degp, h)


def _comb1_body(a1_ref, h_ref, dinv_ref, f1_ref, g2_ref):
    dinv = dinv_ref[...]
    f1 = h_ref[...] - a1_ref[...] * dinv
    f1_ref[...] = f1
    g2_ref[...] = f1 * dinv


def _comb1(a1, h, dinv):
    return pl.pallas_call(
        _comb1_body,
        grid=(3, 10),
        in_specs=[
            pl.BlockSpec((1000, H), lambda c, i: (c * 10 + i, 0)),
            pl.BlockSpec((1000, H), lambda c, i: (i, 0)),
            pl.BlockSpec((1000, 1), lambda c, i: (c * 10 + i, 0)),
        ],
        out_specs=[
            pl.BlockSpec((1000, H), lambda c, i: (c * 10 + i, 0)),
            pl.BlockSpec((1000, H), lambda c, i: (c * 10 + i, 0)),
        ],
        out_shape=[
            jax.ShapeDtypeStruct((C3, H), jnp.float32),
            jax.ShapeDtypeStruct((PAD3, H), jnp.float32),
        ],
    )(a1, h, dinv)


_TH0 = (0.0, 0.0, 3.0)
_TH1 = (0.0, 3.0, -3.0)
_TH2 = (0.75, -1.5, 0.75)


def _final_body(h_ref, f1a_ref, f1b_ref, f1c_ref, a2a_ref, a2b_ref, a2c_ref,
                dva_ref, dvb_ref, dvc_ref, w3_ref, b3_ref, out_ref):
    h = h_ref[...]
    acc = jnp.broadcast_to(b3_ref[...], (1000, H))
    for cc, (f1_ref, a2_ref, dv_ref) in enumerate(
        ((f1a_ref, a2a_ref, dva_ref),
         (f1b_ref, a2b_ref, dvb_ref),
         (f1c_ref, a2c_ref, dvc_ref))
    ):
        f1 = f1_ref[...]
        f2 = f1 - a2_ref[...] * dv_ref[...]
        part = _TH0[cc] * h + _TH1[cc] * f1 + _TH2[cc] * f2
        acc = acc + jnp.dot(part, w3_ref[cc],
                            preferred_element_type=jnp.float32)
    out_ref[...] = jnp.maximum(acc, 0.0)


def _final(h, f1, a2, dinv, W3r, b3r):
    f1_spec = lambda c: pl.BlockSpec((1000, H), lambda i, c=c: (c * 10 + i, 0))
    dv_spec = lambda c: pl.BlockSpec((1000, 1), lambda i, c=c: (c * 10 + i, 0))
    return pl.pallas_call(
        _final_body,
        grid=(10,),
        in_specs=[
            pl.BlockSpec((1000, H), lambda i: (i, 0)),
            f1_spec(0), f1_spec(1), f1_spec(2),
            f1_spec(0), f1_spec(1), f1_spec(2),
            dv_spec(0), dv_spec(1), dv_spec(2),
            pl.BlockSpec((3, H, H), lambda i: (0, 0, 0)),
            pl.BlockSpec((1, H), lambda i: (0, 0)),
        ],
        out_specs=pl.BlockSpec((1000, H), lambda i: (i, 0)),
        out_shape=jax.ShapeDtypeStruct((N, H), jnp.float32),
    )(h, f1, f1, f1, a2, a2, a2, dinv, dinv, dinv, W3r, b3r)


def kernel(feat, edge_index, edge_pred, W1, b1, W2, b2, W3, b3):
    pred2 = edge_pred.reshape(2500, 128)
    src2 = edge_index[0].reshape(2500, 128)
    dst2 = edge_index[1].reshape(2500, 128)
    ks02, ks12, kd02, kd12 = _thresh(pred2, src2, dst2)
    pads = jnp.full((PADE,), SENT, jnp.int32)
    ks0 = jnp.concatenate([ks02.reshape(E), pads]).reshape(NCHUNK, CH)
    ks1 = jnp.concatenate([ks12.reshape(E), pads]).reshape(NCHUNK, CH)
    kd0 = jnp.concatenate([kd02.reshape(E), pads]).reshape(NCHUNK, CH)
    kd1 = jnp.concatenate([kd12.reshape(E), pads]).reshape(NCHUNK, CH)

    h = _mlp(feat, W1, b1.reshape(1, H), W2, b2.reshape(1, H))

    _deg_sc, _hop_sc = _sc_kernels()
    degp = _deg_sc(kd0, kd1)[:, None]
    dinv, G1 = _build(degp, h)
    A1 = _hop_sc(G1, ks0, ks1, kd0, kd1)
    f1, G2 = _comb1(A1, h, dinv)
    A2 = _hop_sc(G2, ks0, ks1, kd0, kd1)
    W3r = W3.reshape(3, H, H)
    return _final(h, f1, A2, dinv, W3r, b3.reshape(1, H))


# speedup vs baseline: 1.1293x; 1.0048x over previous
import functools

import jax
import jax.numpy as jnp
from jax import lax
from jax.experimental import pallas as pl
from jax.experimental.pallas import tpu as pltpu
from jax.experimental.pallas import tpu_sc as plsc

N = 10000
E = 320000
F_IN = 128
H = 64
C3 = 3 * N
NC = 2
NS = 16
PAD3 = 30720
HALF = PAD3 // 2
PADE = 327680 - E
EP = E + PADE
CH = 128
NCHUNK = EP // CH
CPW2 = NCHUNK // NS
ZRH = HALF // NS
ZCH = 120
SENT = 2**31 - 1

_NEG_Q = 0.1
_POS_Q = 0.9


def _thresh_body(pred_ref, src_ref, dst_ref, ks0_ref, ks1_ref, kd0_ref,
                 kd1_ref):
    p = pred_ref[...]
    b = lax.bitcast_convert_type(p, jnp.uint32)
    top = jnp.uint32(0x80000000)
    ful = jnp.uint32(0xFFFFFFFF)
    key = jnp.where(b >= top, ful - b, b + top)

    sel_neg = p <= 0.0
    m_neg = jnp.sum(sel_neg.astype(jnp.int32))
    m_pos = jnp.int32(E) - m_neg

    def ranks(m_i, q):
        m_f = m_i.astype(jnp.float32)
        idx = q * (m_f - 1.0)
        lo_f = jnp.floor(idx)
        hw = idx - lo_f
        t_lo = jnp.clip(lo_f, 0.0, m_f - 1.0).astype(jnp.int32)
        t_hi = jnp.clip(jnp.ceil(idx), 0.0, m_f - 1.0).astype(jnp.int32)
        return t_lo, t_hi, hw

    tn_lo, tn_hi, hw_n = ranks(m_neg, _NEG_Q)
    tp_lo, tp_hi, hw_p = ranks(m_pos, _POS_Q)

    zero = jnp.uint32(0)
    targets = (tn_lo, tn_hi, tp_lo + m_neg, tp_hi + m_neg)

    def bs_body(_, st):
        los, his = st
        new_los, new_his = [], []
        for j in range(4):
            lo, hi = los[j], his[j]
            mid = lo + (hi - lo) // jnp.uint32(2)
            cnt = jnp.sum((key <= mid).astype(jnp.int32))
            good = cnt >= targets[j] + 1
            new_los.append(jnp.where(good, lo, mid + jnp.uint32(1)))
            new_his.append(jnp.where(good, mid, hi))
        return tuple(new_los), tuple(new_his)

    init = ((zero, zero, zero, zero), (ful, ful, ful, ful))
    (los, _) = lax.fori_loop(0, 32, bs_body, init)

    def unkey(k):
        bb = jnp.where(k >= top, k - top, ful - k)
        return lax.bitcast_convert_type(bb, jnp.float32)

    vn_lo, vn_hi, vp_lo, vp_hi = (unkey(k) for k in los)
    neg_thr = jnp.where(m_neg > 0, vn_lo * (1.0 - hw_n) + vn_hi * hw_n, 0.0)
    pos_thr = jnp.where(m_pos > 0, vp_lo * (1.0 - hw_p) + vp_hi * hw_p, 0.0)

    cls = jnp.where(p > pos_thr, 0, jnp.where(p < neg_thr, 2, 1)).astype(jnp.int32)
    ks = cls * N + src_ref[...]
    kd = cls * N + dst_ref[...]
    in0 = kd < HALF
    ks0_ref[...] = jnp.where(in0, ks, SENT)
    ks1_ref[...] = jnp.where(in0, SENT, ks)
    kd0_ref[...] = jnp.where(in0, kd, SENT)
    kd1_ref[...] = jnp.where(in0, SENT, kd - HALF)


def _thresh(pred2, src2, dst2):
    return pl.pallas_call(
        _thresh_body,
        out_shape=[jax.ShapeDtypeStruct((2500, 128), jnp.int32)] * 4,
    )(pred2, src2, dst2)


def _mlp_body(x_ref, w1_ref, b1_ref, w2_ref, b2_ref, h_ref):
    h1 = jnp.maximum(
        jnp.dot(x_ref[...], w1_ref[...], preferred_element_type=jnp.float32)
        + b1_ref[...], 0.0)
    h_ref[...] = jnp.maximum(
        jnp.dot(h1, w2_ref[...], preferred_element_type=jnp.float32)
        + b2_ref[...], 0.0)


def _mlp(feat, W1, b1r, W2, b2r):
    return pl.pallas_call(
        _mlp_body,
        grid=(10,),
        in_specs=[
            pl.BlockSpec((1000, F_IN), lambda i: (i, 0)),
            pl.BlockSpec((F_IN, H), lambda i: (0, 0)),
            pl.BlockSpec((1, H), lambda i: (0, 0)),
            pl.BlockSpec((H, H), lambda i: (0, 0)),
            pl.BlockSpec((1, H), lambda i: (0, 0)),
        ],
        out_specs=pl.BlockSpec((1000, H), lambda i: (i, 0)),
        out_shape=jax.ShapeDtypeStruct((N, H), jnp.float32),
    )(feat, W1, b1r, W2, b2r)


@functools.cache
def _sc_kernels():
    mesh = plsc.VectorSubcoreMesh(core_axis_name="c", subcore_axis_name="s")

    @functools.partial(
        pl.kernel,
        out_type=jax.ShapeDtypeStruct((PAD3,), jnp.float32),
        mesh=mesh,
        compiler_params=pltpu.CompilerParams(use_tc_tiling_on_sc=False),
        scratch_types=[
            pltpu.VMEM((CPW2, CH), jnp.int32),
            pltpu.VMEM((CH,), jnp.float32),
            pltpu.VMEM((ZRH,), jnp.float32),
            pltpu.VMEM_SHARED((HALF,), jnp.float32),
        ],
    )
    def _deg_sc(kd0_hbm, kd1_hbm, out_hbm, di2_v, ones_v, zer_v, acc_sh):
        c = lax.axis_index("c")
        s = lax.axis_index("s")

        def fill(ref, n, val):
            def fb(i, _):
                ref[pl.ds(i * 16, 16)] = jnp.full((16,), val, jnp.float32)
                return 0
            lax.fori_loop(0, n // 16, fb, 0)

        fill(ones_v, CH, 1.0)
        fill(zer_v, ZRH, 0.0)

        @pl.when(c == 0)
        def _():
            pltpu.sync_copy(kd0_hbm.at[pl.ds(s * CPW2, CPW2)], di2_v)

        @pl.when(c == 1)
        def _():
            pltpu.sync_copy(kd1_hbm.at[pl.ds(s * CPW2, CPW2)], di2_v)

        pltpu.sync_copy(zer_v, acc_sh.at[pl.ds(s * ZRH, ZRH)])
        plsc.subcore_barrier()

        def body(i, _):
            pltpu.sync_copy(
                ones_v,
                acc_sh.at[plsc.Indices(di2_v.at[i], ignored_value=SENT)],
                add=True)
            return 0

        lax.fori_loop(0, CPW2, body, 0)
        plsc.subcore_barrier()
        pltpu.sync_copy(acc_sh.at[pl.ds(s * ZRH, ZRH)],
                        out_hbm.at[pl.ds(c * HALF + s * ZRH, ZRH)])

    @functools.partial(
        pl.kernel,
        out_type=jax.ShapeDtypeStruct((PAD3, H), jnp.float32),
        mesh=mesh,
        compiler_params=pltpu.CompilerParams(use_tc_tiling_on_sc=False),
        scratch_types=[
            pltpu.VMEM((CPW2, CH), jnp.int32),
            pltpu.VMEM((CPW2, CH), jnp.int32),
            pltpu.VMEM((CH, H), jnp.float32),
            pltpu.VMEM((CH, H), jnp.float32),
            pltpu.VMEM((ZCH, H), jnp.float32),
            pltpu.VMEM_SHARED((HALF, H), jnp.float32),
            pltpu.SemaphoreType.DMA,
            pltpu.SemaphoreType.DMA,
        ],
    )
    def _hop_sc(gtab_hbm, ks0_hbm, ks1_hbm, kd0_hbm, kd1_hbm, out_hbm,
                si2_v, di2_v, rows0_v, rows1_v, zer_v, acc_sh, sem0, sem1):
        c = lax.axis_index("c")
        s = lax.axis_index("s")
        nz = H // 16

        def zb(i, _):
            zer_v[i // nz, pl.ds((i % nz) * 16, 16)] = jnp.zeros(
                (16,), jnp.float32)
            return 0

        lax.fori_loop(0, ZCH * nz, zb, 0)

        @pl.when(c == 0)
        def _():
            pltpu.sync_copy(ks0_hbm.at[pl.ds(s * CPW2, CPW2)], si2_v)
            pltpu.sync_copy(kd0_hbm.at[pl.ds(s * CPW2, CPW2)], di2_v)

        @pl.when(c == 1)
        def _():
            pltpu.sync_copy(ks1_hbm.at[pl.ds(s * CPW2, CPW2)], si2_v)
            pltpu.sync_copy(kd1_hbm.at[pl.ds(s * CPW2, CPW2)], di2_v)

        def zc(j, _):
            pltpu.sync_copy(zer_v, acc_sh.at[pl.ds(s * ZRH + j * ZCH, ZCH)])
            return 0

        lax.fori_loop(0, ZRH // ZCH, zc, 0)
        plsc.subcore_barrier()

        def gath(i, rows_v, sem):
            pltpu.async_copy(
                gtab_hbm.at[plsc.Indices(si2_v.at[i], ignored_value=SENT)],
                rows_v, sem)

        def wait_rows(rows_v, sem):
            pltpu.make_async_copy(
                gtab_hbm.at[plsc.Indices(si2_v.at[0], ignored_value=SENT)],
                rows_v, sem).wait()

        def scat(rows_v, i):
            pltpu.sync_copy(
                rows_v,
                acc_sh.at[plsc.Indices(di2_v.at[i], ignored_value=SENT)],
                add=True)

        gath(0, rows0_v, sem0)

        def body(j, _):
            i0 = 2 * j
            i1 = i0 + 1
            gath(i1, rows1_v, sem1)
            wait_rows(rows0_v, sem0)
            scat(rows0_v, i0)
            inext = jnp.minimum(i1 + 1, CPW2 - 1)
            gath(inext, rows0_v, sem0)
            wait_rows(rows1_v, sem1)
            scat(rows1_v, i1)
            return 0

        lax.fori_loop(0, CPW2 // 2, body, 0)
        wait_rows(rows0_v, sem0)
        plsc.subcore_barrier()
        pltpu.sync_copy(acc_sh.at[pl.ds(s * ZRH, ZRH)],
                        out_hbm.at[pl.ds(c * HALF + s * ZRH, ZRH)])

    return _deg_sc, _hop_sc


def _build_body(degp_ref, h_ref, dinv_ref, g1_ref):
    dinv = lax.rsqrt(jnp.maximum(degp_ref[...], 1.0))
    g1 = h_ref[...] * dinv
    dinv_ref[...] = jnp.broadcast_to(dinv, (1000, H))
    g1_ref[...] = g1


def _build(degp, h):
    return pl.pallas_call(
        _build_body,
        grid=(3, 10),
        in_specs=[
            pl.BlockSpec((1000, 1), lambda c, i: (c * 10 + i, 0)),
            pl.BlockSpec((1000, H), lambda c, i: (i, 0)),
        ],
        out_specs=[
            pl.BlockSpec((1000, H), lambda c, i: (c * 10 + i, 0)),
            pl.BlockSpec((1000, H), lambda c, i: (c * 10 + i, 0)),
        ],
        out_shape=[
            jax.ShapeDtypeStruct((C3, H), jnp.float32),
            jax.ShapeDtypeStruct((PAD3, H), jnp.float32),
        ],
    )(degp, h)


def _comb1_body(a1_ref, h_ref, dinv_ref, f1_ref, g2_ref):
    dinv = dinv_ref[...]
    f1 = h_ref[...] - a1_ref[...] * dinv
    f1_ref[...] = f1
    g2_ref[...] = f1 * dinv


def _comb1(a1, h, dinv):
    return pl.pallas_call(
        _comb1_body,
        grid=(3, 10),
        in_specs=[
            pl.BlockSpec((1000, H), lambda c, i: (c * 10 + i, 0)),
            pl.BlockSpec((1000, H), lambda c, i: (i, 0)),
            pl.BlockSpec((1000, H), lambda c, i: (c * 10 + i, 0)),
        ],
        out_specs=[
            pl.BlockSpec((1000, H), lambda c, i: (c * 10 + i, 0)),
            pl.BlockSpec((1000, H), lambda c, i: (c * 10 + i, 0)),
        ],
        out_shape=[
            jax.ShapeDtypeStruct((C3, H), jnp.float32),
            jax.ShapeDtypeStruct((PAD3, H), jnp.float32),
        ],
    )(a1, h, dinv)


_TH0 = (0.0, 0.0, 3.0)
_TH1 = (0.0, 3.0, -3.0)
_TH2 = (0.75, -1.5, 0.75)


def _final_body(h_ref, f1a_ref, f1b_ref, f1c_ref, a2a_ref, a2b_ref, a2c_ref,
                dva_ref, dvb_ref, dvc_ref, w3_ref, b3_ref, out_ref):
    h = h_ref[...]
    acc = jnp.broadcast_to(b3_ref[...], (1000, H))
    for cc, (f1_ref, a2_ref, dv_ref) in enumerate(
        ((f1a_ref, a2a_ref, dva_ref),
         (f1b_ref, a2b_ref, dvb_ref),
         (f1c_ref, a2c_ref, dvc_ref))
    ):
        f1 = f1_ref[...]
        f2 = f1 - a2_ref[...] * dv_ref[...]
        part = _TH0[cc] * h + _TH1[cc] * f1 + _TH2[cc] * f2
        acc = acc + jnp.dot(part, w3_ref[cc],
                            preferred_element_type=jnp.float32)
    out_ref[...] = jnp.maximum(acc, 0.0)


def _final(h, f1, a2, dinv, W3r, b3r):
    f1_spec = lambda c: pl.BlockSpec((1000, H), lambda i, c=c: (c * 10 + i, 0))
    dv_spec = lambda c: pl.BlockSpec((1000, H), lambda i, c=c: (c * 10 + i, 0))
    return pl.pallas_call(
        _final_body,
        grid=(10,),
        in_specs=[
            pl.BlockSpec((1000, H), lambda i: (i, 0)),
            f1_spec(0), f1_spec(1), f1_spec(2),
            f1_spec(0), f1_spec(1), f1_spec(2),
            dv_spec(0), dv_spec(1), dv_spec(2),
            pl.BlockSpec((3, H, H), lambda i: (0, 0, 0)),
            pl.BlockSpec((1, H), lambda i: (0, 0)),
        ],
        out_specs=pl.BlockSpec((1000, H), lambda i: (i, 0)),
        out_shape=jax.ShapeDtypeStruct((N, H), jnp.float32),
    )(h, f1, f1, f1, a2, a2, a2, dinv, dinv, dinv, W3r, b3r)


def kernel(feat, edge_index, edge_pred, W1, b1, W2, b2, W3, b3):
    pred2 = edge_pred.reshape(2500, 128)
    src2 = edge_index[0].reshape(2500, 128)
    dst2 = edge_index[1].reshape(2500, 128)
    ks02, ks12, kd02, kd12 = _thresh(pred2, src2, dst2)
    pads = jnp.full((PADE,), SENT, jnp.int32)
    ks0 = jnp.concatenate([ks02.reshape(E), pads]).reshape(NCHUNK, CH)
    ks1 = jnp.concatenate([ks12.reshape(E), pads]).reshape(NCHUNK, CH)
    kd0 = jnp.concatenate([kd02.reshape(E), pads]).reshape(NCHUNK, CH)
    kd1 = jnp.concatenate([kd12.reshape(E), pads]).reshape(NCHUNK, CH)

    h = _mlp(feat, W1, b1.reshape(1, H), W2, b2.reshape(1, H))

    _deg_sc, _hop_sc = _sc_kernels()
    degp = _deg_sc(kd0, kd1)[:, None]
    dinv, G1 = _build(degp, h)
    A1 = _hop_sc(G1, ks0, ks1, kd0, kd1)
    f1, G2 = _comb1(A1, h, dinv)
    A2 = _hop_sc(G2, ks0, ks1, kd0, kd1)
    W3r = W3.reshape(3, H, H)
    return _final(h, f1, A2, dinv, W3r, b3.reshape(1, H))
